# baseline (speedup 1.0000x reference)
import functools

import jax
import jax.numpy as jnp
import numpy as np
from jax import lax
from jax.experimental import pallas as pl
from jax.experimental.pallas import tpu as pltpu
from jax.experimental.pallas import tpu_sc as plsc

N = 10000
E = 160000
D = 128
H = 8
DK = 16
DNT = (1, 0, 2)
EPERM = (1, 0, 2)

BLK = 400
NBLK = N // BLK

NTILE = 16
E_PER_TILE = E // NTILE
CHUNK = 32
NFULL = E_PER_TILE // CHUNK
TAIL = E_PER_TILE - NFULL * CHUNK
GROUP = 4
NGROUP = NFULL // GROUP
RCHUNK = 16
NRCH = N // RCHUNK
NRCH_PER_TILE = (NRCH + NTILE - 1) // NTILE


def _build_perm():
    perm = np.zeros(128, dtype=np.int32)
    for c in range(128):
        j, r = divmod(c, 16)
        p, hh = divmod(r, 8)
        h = hh if p == 0 else 7 - hh
        d = 2 * j + p
        perm[c] = h * 16 + d
    return perm


_PERM = _build_perm()
_HOL = np.array([r % 8 if r < 8 else 7 - (r % 8) for r in range(16)],
                dtype=np.int32)


def _stage1_body(x0, x1, x2, ct, A_k, b_k, A_v, b_v, A_m, b_m, A_q, b_q,
                 kvtab, qtab):
    x0b = x0[...]
    kd = jnp.dot(x0b, A_k[...], preferred_element_type=jnp.float32) + b_k[...]
    vd = jnp.dot(x0b, A_v[...], preferred_element_type=jnp.float32) + b_v[...]
    ctb = ct[...]
    md = jnp.zeros_like(kd)
    for t in range(3):
        mm = jnp.dot(x0b, A_m[t], preferred_element_type=jnp.float32) + b_m[t, 0:1, :]
        md = md + jnp.where(ctb == t, mm, 0.0)
    kvtab[0:1, :, 0:128] = kd[None]
    kvtab[0:1, :, 128:256] = vd[None]
    kvtab[1:2, :, 0:128] = md[None]
    kvtab[1:2, :, 128:256] = vd[None]
    xs = (x1[...], x0b, x2[...])
    for e in range(3):
        q = jnp.dot(xs[e], A_q[e], preferred_element_type=jnp.float32) + b_q[e, 0:1, :]
        qtab[e:e + 1] = q[None]


def _stage1(x0, x1, x2, ct, A_k, b_k, A_v, b_v, A_m, b_m, A_q, b_q):
    full = lambda s: pl.BlockSpec(s, lambda i: (0,) * len(s))
    row = lambda s: pl.BlockSpec(s, lambda i: (i,) + (0,) * (len(s) - 1))
    return pl.pallas_call(
        _stage1_body,
        grid=(NBLK,),
        in_specs=[
            row((BLK, 128)), row((BLK, 128)), row((BLK, 128)), row((BLK, 128)),
            full((128, 128)), full((1, 128)),
            full((128, 128)), full((1, 128)),
            full((3, 128, 128)), full((3, 1, 128)),
            full((3, 128, 128)), full((3, 1, 128)),
        ],
        out_specs=[
            pl.BlockSpec((2, BLK, 256), lambda i: (0, i, 0)),
            pl.BlockSpec((3, BLK, 128), lambda i: (0, i, 0)),
        ],
        out_shape=[
            jax.ShapeDtypeStruct((2, N, 256), jnp.float32),
            jax.ShapeDtypeStruct((3, N, 128), jnp.float32),
        ],
    )(x0, x1, x2, ct, A_k, b_k, A_v, b_v, A_m, b_m, A_q, b_q)


def _sc_body(kvtab_hbm, qtab_hbm, src_hbm, dst_hbm, scales_hbm, out_hbm,
             sidx0, dstv0, qidx0, kvbuf0, qrows0, outb0,
             sidx1, dstv1, qidx1, kvbuf1, qrows1, outb1,
             sblkA, dblkA, sblkB, dblkB,
             sidxt, dstvt, qidxt, scalev, sem0, sem1, ssem0, ssem1,
             acc):
    br = lax.axis_index("c")
    tl = lax.axis_index("s")
    sets = ((sidx0, dstv0, qidx0, kvbuf0, qrows0, outb0, sem0, ssem0),
            (sidx1, dstv1, qidx1, kvbuf1, qrows1, outb1, sem1, ssem1))
    blks = ((sblkA, dblkA), (sblkB, dblkB))

    for e in range(3):
        def zrow(i, carry):
            r = i // 9
            col = (i % 9) * 16
            outb0[r, pl.ds(col, 16)] = jnp.zeros((16,), jnp.float32)
            return carry
        lax.fori_loop(0, RCHUNK * 9, zrow, 0)

        def zinit(i, carry):
            c = jnp.minimum(i * NTILE + tl, NRCH - 1)
            off = pl.multiple_of(c * RCHUNK, RCHUNK)
            pltpu.sync_copy(outb0.at[pl.ds(0, RCHUNK)],
                            acc.at[pl.ds(off, RCHUNK)])
            return carry
        lax.fori_loop(0, NRCH_PER_TILE, zinit, 0)
        soff = pl.multiple_of((br * 3 + e) * 16, 16)
        pltpu.sync_copy(scales_hbm.at[pl.ds(soff, 16)], scalev)
        plsc.subcore_barrier()

        def blkload(g, pair):
            gg = jnp.minimum(g, NGROUP - 1)
            off = pl.multiple_of(
                e * E + tl * E_PER_TILE + gg * (GROUP * CHUNK), 16)
            pltpu.sync_copy(src_hbm.at[pl.ds(off, GROUP * CHUNK)], pair[0])
            pltpu.sync_copy(dst_hbm.at[pl.ds(off, GROUP * CHUNK)], pair[1])

        def prep(j, s, pair, slot, first=False):
            sidx, dstv, qidx, kvbuf, qrows, outb, sem, ssem = s
            if not first:
                @pl.when(j >= 2)
                def _():
                    pltpu.make_async_copy(outb, acc.at[dstv], ssem).wait()
            sblk, dblk = pair
            for t in range(CHUNK // 16):
                col = slot * CHUNK + t * 16
                dv = dblk[pl.ds(col, 16)]
                sidx[pl.ds(t * 16, 16)] = sblk[pl.ds(col, 16)] + br * N
                qidx[pl.ds(t * 16, 16)] = dv + e * N
                dstv[pl.ds(t * 16, 16)] = dv
            pltpu.make_async_copy(kvtab_hbm.at[sidx], kvbuf, sem).start()
            pltpu.make_async_copy(qtab_hbm.at[qidx], qrows, sem).start()

        def consume(s):
            sidx, dstv, qidx, kvbuf, qrows, outb, sem, ssem = s
            pltpu.make_async_copy(kvtab_hbm.at[sidx], kvbuf, sem).wait()
            pltpu.make_async_copy(qtab_hbm.at[qidx], qrows, sem).wait()
            sc = scalev[...]

            def eb(c, c2):
                a = kvbuf[c, pl.ds(0, 16)] * qrows[c, pl.ds(0, 16)]
                for j in range(1, 8):
                    a = a + kvbuf[c, pl.ds(j * 16, 16)] * qrows[c, pl.ds(j * 16, 16)]
                a = a + lax.rev(a, (0,))
                ex = jnp.exp(jnp.minimum(a * sc, 60.0))
                outb[c, pl.ds(128, 16)] = ex
                for j in range(8):
                    outb[c, pl.ds(j * 16, 16)] = ex * kvbuf[c, pl.ds(128 + j * 16, 16)]
                return c2
            lax.fori_loop(0, CHUNK, eb, 0)
            pltpu.make_async_copy(outb, acc.at[dstv], ssem).start(add=True)

        blkload(0, blks[0])
        prep(0, sets[0], blks[0], 0, first=True)

        @pl.loop(0, NGROUP, step=2)
        def _ring(go):
            for gp in range(2):
                gi = go + gp
                cur = blks[gp]
                nxt = blks[1 - gp]
                blkload(gi + 1, nxt)
                for b in range(GROUP):
                    j = gi * GROUP + b
                    s = sets[b % 2]
                    npair = cur if b < GROUP - 1 else nxt

                    @pl.when(j + 1 < NFULL)
                    def _(npair=npair, b=b, j=j):
                        prep(j + 1, sets[(b + 1) % 2], npair, (b + 1) % GROUP)
                    consume(s)

        for s in sets:
            _, dstv_s, _, _, _, outb_s, _, ssem_s = s
            pltpu.make_async_copy(outb_s, acc.at[dstv_s], ssem_s).wait()

        toff = pl.multiple_of(e * E + tl * E_PER_TILE + NFULL * CHUNK, 16)
        pltpu.sync_copy(src_hbm.at[pl.ds(toff, TAIL)], sidxt)
        pltpu.sync_copy(dst_hbm.at[pl.ds(toff, TAIL)], dstvt)
        sidxt[pl.ds(0, 16)] = sidxt[pl.ds(0, 16)] + br * N
        qidxt[pl.ds(0, 16)] = dstvt[pl.ds(0, 16)] + e * N
        pltpu.sync_copy(kvtab_hbm.at[sidxt], kvbuf0.at[pl.ds(0, TAIL)])
        pltpu.sync_copy(qtab_hbm.at[qidxt], qrows0.at[pl.ds(0, TAIL)])
        sct = scalev[...]

        def ebt(c, c2):
            a = kvbuf0[c, pl.ds(0, 16)] * qrows0[c, pl.ds(0, 16)]
            for j in range(1, 8):
                a = a + kvbuf0[c, pl.ds(j * 16, 16)] * qrows0[c, pl.ds(j * 16, 16)]
            a = a + lax.rev(a, (0,))
            ex = jnp.exp(jnp.minimum(a * sct, 60.0))
            outb0[c, pl.ds(128, 16)] = ex
            for j in range(8):
                outb0[c, pl.ds(j * 16, 16)] = ex * kvbuf0[c, pl.ds(128 + j * 16, 16)]
            return c2
        lax.fori_loop(0, TAIL, ebt, 0)
        pltpu.sync_copy(outb0.at[pl.ds(0, TAIL)], acc.at[dstvt], add=True)
        plsc.subcore_barrier()

        def cout(i, carry):
            c = jnp.minimum(i * NTILE + tl, NRCH - 1)
            off = pl.multiple_of(c * RCHUNK, RCHUNK)
            pltpu.sync_copy(acc.at[pl.ds(off, RCHUNK)],
                            out_hbm.at[br, EPERM[e], pl.ds(off, RCHUNK)])
            return carry
        lax.fori_loop(0, NRCH_PER_TILE, cout, 0)
        plsc.subcore_barrier()


@functools.partial(jax.jit, static_argnums=())
def _sc_edge(kvtab, qtab, src, dst, scales):
    mesh = plsc.VectorSubcoreMesh(core_axis_name="c", subcore_axis_name="s")
    bufset = [
        pltpu.VMEM((CHUNK,), jnp.int32),
        pltpu.VMEM((CHUNK,), jnp.int32),
        pltpu.VMEM((CHUNK,), jnp.int32),
        pltpu.VMEM((CHUNK, 256), jnp.float32),
        pltpu.VMEM((CHUNK, 128), jnp.float32),
        pltpu.VMEM((CHUNK, 144), jnp.float32),
    ]
    f = pl.kernel(
        _sc_body,
        out_type=[
            jax.ShapeDtypeStruct((2, 3, N, 144), jnp.float32),
        ],
        mesh=mesh,
        compiler_params=pltpu.CompilerParams(use_tc_tiling_on_sc=False),
        scratch_types=bufset + bufset + [
            pltpu.VMEM((GROUP * CHUNK,), jnp.int32),
            pltpu.VMEM((GROUP * CHUNK,), jnp.int32),
            pltpu.VMEM((GROUP * CHUNK,), jnp.int32),
            pltpu.VMEM((GROUP * CHUNK,), jnp.int32),
            pltpu.VMEM((TAIL,), jnp.int32),
            pltpu.VMEM((TAIL,), jnp.int32),
            pltpu.VMEM((TAIL,), jnp.int32),
            pltpu.VMEM((16,), jnp.float32),
            pltpu.SemaphoreType.DMA,
            pltpu.SemaphoreType.DMA,
            pltpu.SemaphoreType.DMA,
            pltpu.SemaphoreType.DMA,
            pltpu.VMEM_SHARED((N, 144), jnp.float32),
        ],
    )
    return f(kvtab, qtab, src, dst, scales)


def _stage3_body(acc_r, G, Gc, te, comb, out):
    accb = acc_r[0, 0]
    num = accb[:, 0:128]
    den16 = accb[:, 128:144]
    den = jnp.tile(den16, (1, 8))
    den = jnp.where(den == 0.0, 1.0, den)
    h = jnp.dot(num / den, G[0], preferred_element_type=jnp.float32)
    caccb = acc_r[1, 0]
    cnum = caccb[:, 0:128]
    cden16 = caccb[:, 128:144]
    cden = jnp.tile(cden16, (1, 8))
    cdeng = jnp.where(cden == 0.0, 1.0, cden)
    teb = te[...]
    ch = jnp.dot((cnum + cden * teb) / cdeng, Gc[0],
                 preferred_element_type=jnp.float32)
    out[0] = jnp.maximum(h + ch * comb[0], 0.0)


def _stage3(acc, G_p, Gc_p, te_dm, comb_p):
    return pl.pallas_call(
        _stage3_body,
        grid=(3, NBLK),
        in_specs=[
            pl.BlockSpec((2, 1, BLK, 144), lambda e, i: (0, e, i, 0)),
            pl.BlockSpec((1, 128, 128), lambda e, i: (e, 0, 0)),
            pl.BlockSpec((1, 128, 128), lambda e, i: (e, 0, 0)),
            pl.BlockSpec((1, 128), lambda e, i: (0, 0)),
            pl.BlockSpec((1, 1, 128), lambda e, i: (e, 0, 0)),
        ],
        out_specs=pl.BlockSpec((1, BLK, 128), lambda e, i: (e, i, 0)),
        out_shape=jax.ShapeDtypeStruct((3, N, 128), jnp.float32),
    )(acc, G_p, Gc_p, te_dm, comb_p)


def _blockdiag(rel, perm):
    out = jnp.zeros((3, 128, 128), dtype=jnp.float32)
    for e in range(3):
        for h in range(H):
            out = out.at[e, h * 16:(h + 1) * 16, h * 16:(h + 1) * 16].set(rel[e, h])
    return out[:, perm, :]


def kernel(x, Wk, bk, Wq, bq, Wv, bv, rel_pri, rel_msg, rel_pri_cau,
           rel_msg_cau, comb_pri, cau_filter, time_emb, src_idx, dst_idx,
           cau_type):
    perm = jnp.asarray(_PERM)
    hol = jnp.asarray(_HOL)

    A_k = Wk[0].T[:, perm]
    b_k = bk[0][perm][None, :]
    A_v = Wv[0].T[:, perm]
    b_v = bv[0][perm][None, :]
    Mbd = _blockdiag(cau_filter, jnp.arange(128))
    A_m = jnp.einsum('ab,tbc->tac', Wk[0].T, Mbd[:, :, perm])
    b_m = jnp.einsum('b,tbc->tc', bk[0], Mbd[:, :, perm])[:, None, :]
    A_q = jnp.stack([Wq[DNT[e]].T[:, perm] for e in range(3)])
    b_q = jnp.stack([bq[DNT[e]][perm] for e in range(3)])[:, None, :]
    G = _blockdiag(rel_msg, perm)
    Gc = _blockdiag(rel_msg_cau, perm)
    G_p = G[jnp.asarray(EPERM)]
    Gc_p = Gc[jnp.asarray(EPERM)]
    te_hm = jnp.transpose(time_emb, (1, 0, 2)).reshape(128)
    te_dm = te_hm[perm][None, :]
    comb_hm = comb_pri.reshape(3, 128)
    comb_p = comb_hm[jnp.asarray(EPERM)][:, None, :]
    scales = jnp.stack([rel_pri[:, hol] / 4.0, rel_pri_cau[:, hol] / 4.0])

    ct = jnp.broadcast_to(cau_type[:, None], (N, 128)).astype(jnp.int32)

    kvtab, qtab = _stage1(x[0], x[1], x[2], ct, A_k, b_k, A_v, b_v,
                          A_m, b_m, A_q, b_q)
    (acc,) = _sc_edge(kvtab.reshape(2 * N, 256), qtab.reshape(3 * N, 128),
                      src_idx.reshape(3 * E), dst_idx.reshape(3 * E),
                      scales.reshape(96))
    return _stage3(acc, G_p, Gc_p, te_dm, comb_p)


# speedup vs baseline: 1.6286x; 1.6286x over previous
import functools

import jax
import jax.numpy as jnp
import numpy as np
from jax import lax
from jax.experimental import pallas as pl
from jax.experimental.pallas import tpu as pltpu
from jax.experimental.pallas import tpu_sc as plsc

N = 10000
E = 160000
D = 128
H = 8
DK = 16
DNT = (1, 0, 2)
EPERM = (1, 0, 2)

BLK = 400
NBLK = N // BLK

NTILE = 16
E_PER_TILE = E // NTILE
CHUNK = 48
NFULL = E_PER_TILE // CHUNK
TAIL = E_PER_TILE - NFULL * CHUNK
GROUP = 4
NGROUP = NFULL // GROUP
RCHUNK = 16
NRCH = N // RCHUNK
NRCH_PER_TILE = (NRCH + NTILE - 1) // NTILE


def _build_perm():
    perm = np.zeros(128, dtype=np.int32)
    for c in range(128):
        j, r = divmod(c, 16)
        p, hh = divmod(r, 8)
        h = hh if p == 0 else 7 - hh
        d = 2 * j + p
        perm[c] = h * 16 + d
    return perm


_PERM = _build_perm()
_HOL = np.array([r % 8 if r < 8 else 7 - (r % 8) for r in range(16)],
                dtype=np.int32)


def _stage1_body(x0, x1, x2, ct, A_k, b_k, A_v, b_v, A_m, b_m, A_q, b_q,
                 ktab, vtab, qtab):
    x0b = x0[...]
    kd = jnp.dot(x0b, A_k[...], preferred_element_type=jnp.float32) + b_k[...]
    vd = jnp.dot(x0b, A_v[...], preferred_element_type=jnp.float32) + b_v[...]
    ctb = ct[...]
    md = jnp.zeros_like(kd)
    for t in range(3):
        mm = jnp.dot(x0b, A_m[t], preferred_element_type=jnp.float32) + b_m[t, 0:1, :]
        md = md + jnp.where(ctb == t, mm, 0.0)
    ktab[0:1] = kd[None]
    ktab[1:2] = md[None]
    vtab[0:1] = vd[None]
    vtab[1:2] = vd[None]
    xs = (x1[...], x0b, x2[...])
    for e in range(3):
        q = jnp.dot(xs[e], A_q[e], preferred_element_type=jnp.float32) + b_q[e, 0:1, :]
        qtab[e:e + 1] = q[None]


def _stage1(x0, x1, x2, ct, A_k, b_k, A_v, b_v, A_m, b_m, A_q, b_q):
    full = lambda s: pl.BlockSpec(s, lambda i: (0,) * len(s))
    row = lambda s: pl.BlockSpec(s, lambda i: (i,) + (0,) * (len(s) - 1))
    return pl.pallas_call(
        _stage1_body,
        grid=(NBLK,),
        in_specs=[
            row((BLK, 128)), row((BLK, 128)), row((BLK, 128)), row((BLK, 128)),
            full((128, 128)), full((1, 128)),
            full((128, 128)), full((1, 128)),
            full((3, 128, 128)), full((3, 1, 128)),
            full((3, 128, 128)), full((3, 1, 128)),
        ],
        out_specs=[
            pl.BlockSpec((2, BLK, 128), lambda i: (0, i, 0)),
            pl.BlockSpec((2, BLK, 128), lambda i: (0, i, 0)),
            pl.BlockSpec((3, BLK, 128), lambda i: (0, i, 0)),
        ],
        out_shape=[
            jax.ShapeDtypeStruct((2, N, 128), jnp.float32),
            jax.ShapeDtypeStruct((2, N, 128), jnp.float32),
            jax.ShapeDtypeStruct((3, N, 128), jnp.float32),
        ],
    )(x0, x1, x2, ct, A_k, b_k, A_v, b_v, A_m, b_m, A_q, b_q)


def _sc_body(ktab_hbm, vtab_hbm, qtab_hbm, src_hbm, dst_hbm, scales_hbm,
             outn_hbm, outd_hbm,
             sidx0, dstv0, qidx0, kbuf0, qrows0, outbv0, exb0,
             sidx1, dstv1, qidx1, kbuf1, qrows1, outbv1, exb1,
             sblkA, dblkA, sblkB, dblkB,
             sidxt, dstvt, qidxt, scalev, sem0, sem1, ssem0, ssem1,
             numacc, denacc):
    br = lax.axis_index("c")
    tl = lax.axis_index("s")
    sets = ((sidx0, dstv0, qidx0, kbuf0, qrows0, outbv0, exb0, sem0, ssem0),
            (sidx1, dstv1, qidx1, kbuf1, qrows1, outbv1, exb1, sem1, ssem1))
    blks = ((sblkA, dblkA), (sblkB, dblkB))

    for e in range(3):
        def zrow(i, carry):
            r = i // 8
            col = (i % 8) * 16
            outbv0[r, pl.ds(col, 16)] = jnp.zeros((16,), jnp.float32)
            return carry
        lax.fori_loop(0, RCHUNK * 8, zrow, 0)

        def zrow2(i, carry):
            exb0[i, pl.ds(0, 16)] = jnp.zeros((16,), jnp.float32)
            return carry
        lax.fori_loop(0, RCHUNK, zrow2, 0)

        def zinit(i, carry):
            c = jnp.minimum(i * NTILE + tl, NRCH - 1)
            off = pl.multiple_of(c * RCHUNK, RCHUNK)
            pltpu.sync_copy(outbv0.at[pl.ds(0, RCHUNK)],
                            numacc.at[pl.ds(off, RCHUNK)])
            pltpu.sync_copy(exb0.at[pl.ds(0, RCHUNK)],
                            denacc.at[pl.ds(off, RCHUNK)])
            return carry
        lax.fori_loop(0, NRCH_PER_TILE, zinit, 0)
        soff = pl.multiple_of((br * 3 + e) * 16, 16)
        pltpu.sync_copy(scales_hbm.at[pl.ds(soff, 16)], scalev)
        plsc.subcore_barrier()

        def blkload(g, pair):
            gg = jnp.minimum(g, NGROUP - 1)
            off = pl.multiple_of(
                e * E + tl * E_PER_TILE + gg * (GROUP * CHUNK), 16)
            pltpu.sync_copy(src_hbm.at[pl.ds(off, GROUP * CHUNK)], pair[0])
            pltpu.sync_copy(dst_hbm.at[pl.ds(off, GROUP * CHUNK)], pair[1])

        def prep(j, s, pair, slot, first=False):
            sidx, dstv, qidx, kbuf, qrows, outbv, exb, sem, ssem = s
            if not first:
                @pl.when(j >= 2)
                def _():
                    pltpu.make_async_copy(outbv, numacc.at[dstv], ssem).wait()
                    pltpu.make_async_copy(exb, denacc.at[dstv], ssem).wait()
            sblk, dblk = pair
            for t in range(CHUNK // 16):
                col = slot * CHUNK + t * 16
                dv = dblk[pl.ds(col, 16)]
                sidx[pl.ds(t * 16, 16)] = sblk[pl.ds(col, 16)] + br * N
                qidx[pl.ds(t * 16, 16)] = dv + e * N
                dstv[pl.ds(t * 16, 16)] = dv
            pltpu.make_async_copy(vtab_hbm.at[sidx], outbv, sem).start()
            pltpu.make_async_copy(ktab_hbm.at[sidx], kbuf, sem).start()
            pltpu.make_async_copy(qtab_hbm.at[qidx], qrows, sem).start()

        def consume(s):
            sidx, dstv, qidx, kbuf, qrows, outbv, exb, sem, ssem = s
            pltpu.make_async_copy(vtab_hbm.at[sidx], outbv, sem).wait()
            pltpu.make_async_copy(ktab_hbm.at[sidx], kbuf, sem).wait()
            pltpu.make_async_copy(qtab_hbm.at[qidx], qrows, sem).wait()
            sc = scalev[...]

            def eb(i, c2):
                c0 = i * 2
                for c in (c0, c0 + 1):
                    a = kbuf[c, pl.ds(0, 16)] * qrows[c, pl.ds(0, 16)]
                    for j in range(1, 8):
                        a = a + kbuf[c, pl.ds(j * 16, 16)] * qrows[c, pl.ds(j * 16, 16)]
                    a = a + lax.rev(a, (0,))
                    ex = jnp.exp(jnp.minimum(a * sc, 60.0))
                    exb[c, pl.ds(0, 16)] = ex
                    for j in range(8):
                        outbv[c, pl.ds(j * 16, 16)] = ex * outbv[c, pl.ds(j * 16, 16)]
                return c2
            lax.fori_loop(0, CHUNK // 2, eb, 0)
            pltpu.make_async_copy(outbv, numacc.at[dstv], ssem).start(add=True)
            pltpu.make_async_copy(exb, denacc.at[dstv], ssem).start(add=True)

        blkload(0, blks[0])
        prep(0, sets[0], blks[0], 0, first=True)

        @pl.loop(0, NGROUP, step=2)
        def _ring(go):
            for gp in range(2):
                gi = go + gp
                cur = blks[gp]
                nxt = blks[1 - gp]
                blkload(gi + 1, nxt)
                for b in range(GROUP):
                    j = gi * GROUP + b
                    s = sets[b % 2]
                    npair = cur if b < GROUP - 1 else nxt

                    @pl.when(j + 1 < NFULL)
                    def _(npair=npair, b=b, j=j):
                        prep(j + 1, sets[(b + 1) % 2], npair, (b + 1) % GROUP)
                    consume(s)

        for s in sets:
            _, dstv_s, _, _, _, outbv_s, exb_s, _, ssem_s = s
            pltpu.make_async_copy(outbv_s, numacc.at[dstv_s], ssem_s).wait()
            pltpu.make_async_copy(exb_s, denacc.at[dstv_s], ssem_s).wait()

        toff = pl.multiple_of(e * E + tl * E_PER_TILE + NFULL * CHUNK, 16)
        pltpu.sync_copy(src_hbm.at[pl.ds(toff, TAIL)], sidxt)
        pltpu.sync_copy(dst_hbm.at[pl.ds(toff, TAIL)], dstvt)
        sidxt[pl.ds(0, 16)] = sidxt[pl.ds(0, 16)] + br * N
        qidxt[pl.ds(0, 16)] = dstvt[pl.ds(0, 16)] + e * N
        pltpu.sync_copy(vtab_hbm.at[sidxt], outbv0.at[pl.ds(0, TAIL)])
        pltpu.sync_copy(ktab_hbm.at[sidxt], kbuf0.at[pl.ds(0, TAIL)])
        pltpu.sync_copy(qtab_hbm.at[qidxt], qrows0.at[pl.ds(0, TAIL)])
        sct = scalev[...]

        def ebt(c, c2):
            a = kbuf0[c, pl.ds(0, 16)] * qrows0[c, pl.ds(0, 16)]
            for j in range(1, 8):
                a = a + kbuf0[c, pl.ds(j * 16, 16)] * qrows0[c, pl.ds(j * 16, 16)]
            a = a + lax.rev(a, (0,))
            ex = jnp.exp(jnp.minimum(a * sct, 60.0))
            exb0[c, pl.ds(0, 16)] = ex
            for j in range(8):
                outbv0[c, pl.ds(j * 16, 16)] = ex * outbv0[c, pl.ds(j * 16, 16)]
            return c2
        lax.fori_loop(0, TAIL, ebt, 0)
        pltpu.sync_copy(outbv0.at[pl.ds(0, TAIL)], numacc.at[dstvt], add=True)
        pltpu.sync_copy(exb0.at[pl.ds(0, TAIL)], denacc.at[dstvt], add=True)
        plsc.subcore_barrier()

        def cout(i, carry):
            c = jnp.minimum(i * NTILE + tl, NRCH - 1)
            off = pl.multiple_of(c * RCHUNK, RCHUNK)
            pltpu.sync_copy(numacc.at[pl.ds(off, RCHUNK)],
                            outn_hbm.at[br, EPERM[e], pl.ds(off, RCHUNK)])
            pltpu.sync_copy(denacc.at[pl.ds(off, RCHUNK)],
                            outd_hbm.at[br, EPERM[e], pl.ds(off, RCHUNK)])
            return carry
        lax.fori_loop(0, NRCH_PER_TILE, cout, 0)
        plsc.subcore_barrier()


@functools.partial(jax.jit, static_argnums=())
def _sc_edge(ktab, vtab, qtab, src, dst, scales):
    mesh = plsc.VectorSubcoreMesh(core_axis_name="c", subcore_axis_name="s")
    bufset = [
        pltpu.VMEM((CHUNK,), jnp.int32),
        pltpu.VMEM((CHUNK,), jnp.int32),
        pltpu.VMEM((CHUNK,), jnp.int32),
        pltpu.VMEM((CHUNK, 128), jnp.float32),
        pltpu.VMEM((CHUNK, 128), jnp.float32),
        pltpu.VMEM((CHUNK, 128), jnp.float32),
        pltpu.VMEM((CHUNK, 16), jnp.float32),
    ]
    f = pl.kernel(
        _sc_body,
        out_type=[
            jax.ShapeDtypeStruct((2, 3, N, 128), jnp.float32),
            jax.ShapeDtypeStruct((2, 3, N, 16), jnp.float32),
        ],
        mesh=mesh,
        compiler_params=pltpu.CompilerParams(use_tc_tiling_on_sc=False),
        scratch_types=bufset + bufset + [
            pltpu.VMEM((GROUP * CHUNK,), jnp.int32),
            pltpu.VMEM((GROUP * CHUNK,), jnp.int32),
            pltpu.VMEM((GROUP * CHUNK,), jnp.int32),
            pltpu.VMEM((GROUP * CHUNK,), jnp.int32),
            pltpu.VMEM((TAIL,), jnp.int32),
            pltpu.VMEM((TAIL,), jnp.int32),
            pltpu.VMEM((TAIL,), jnp.int32),
            pltpu.VMEM((16,), jnp.float32),
            pltpu.SemaphoreType.DMA,
            pltpu.SemaphoreType.DMA,
            pltpu.SemaphoreType.DMA,
            pltpu.SemaphoreType.DMA,
            pltpu.VMEM_SHARED((N, 128), jnp.float32),
            pltpu.VMEM_SHARED((N, 16), jnp.float32),
        ],
    )
    return f(ktab, vtab, qtab, src, dst, scales)


def _stage3_body(accn_r, accd_r, G, Gc, te, comb, out):
    num = accn_r[0, 0]
    den16 = accd_r[0, 0]
    den = jnp.tile(den16, (1, 8))
    den = jnp.where(den == 0.0, 1.0, den)
    h = jnp.dot(num / den, G[0], preferred_element_type=jnp.float32)
    cnum = accn_r[1, 0]
    cden16 = accd_r[1, 0]
    cden = jnp.tile(cden16, (1, 8))
    cdeng = jnp.where(cden == 0.0, 1.0, cden)
    teb = te[...]
    ch = jnp.dot((cnum + cden * teb) / cdeng, Gc[0],
                 preferred_element_type=jnp.float32)
    out[0] = jnp.maximum(h + ch * comb[0], 0.0)


def _stage3(accn, accd, G_p, Gc_p, te_dm, comb_p):
    return pl.pallas_call(
        _stage3_body,
        grid=(3, NBLK),
        in_specs=[
            pl.BlockSpec((2, 1, BLK, 128), lambda e, i: (0, e, i, 0)),
            pl.BlockSpec((2, 1, BLK, 16), lambda e, i: (0, e, i, 0)),
            pl.BlockSpec((1, 128, 128), lambda e, i: (e, 0, 0)),
            pl.BlockSpec((1, 128, 128), lambda e, i: (e, 0, 0)),
            pl.BlockSpec((1, 128), lambda e, i: (0, 0)),
            pl.BlockSpec((1, 1, 128), lambda e, i: (e, 0, 0)),
        ],
        out_specs=pl.BlockSpec((1, BLK, 128), lambda e, i: (e, i, 0)),
        out_shape=jax.ShapeDtypeStruct((3, N, 128), jnp.float32),
    )(accn, accd, G_p, Gc_p, te_dm, comb_p)


def _blockdiag(rel, perm):
    out = jnp.zeros((3, 128, 128), dtype=jnp.float32)
    for e in range(3):
        for h in range(H):
            out = out.at[e, h * 16:(h + 1) * 16, h * 16:(h + 1) * 16].set(rel[e, h])
    return out[:, perm, :]


def kernel(x, Wk, bk, Wq, bq, Wv, bv, rel_pri, rel_msg, rel_pri_cau,
           rel_msg_cau, comb_pri, cau_filter, time_emb, src_idx, dst_idx,
           cau_type):
    perm = jnp.asarray(_PERM)
    hol = jnp.asarray(_HOL)

    A_k = Wk[0].T[:, perm]
    b_k = bk[0][perm][None, :]
    A_v = Wv[0].T[:, perm]
    b_v = bv[0][perm][None, :]
    Mbd = _blockdiag(cau_filter, jnp.arange(128))
    A_m = jnp.einsum('ab,tbc->tac', Wk[0].T, Mbd[:, :, perm])
    b_m = jnp.einsum('b,tbc->tc', bk[0], Mbd[:, :, perm])[:, None, :]
    A_q = jnp.stack([Wq[DNT[e]].T[:, perm] for e in range(3)])
    b_q = jnp.stack([bq[DNT[e]][perm] for e in range(3)])[:, None, :]
    G = _blockdiag(rel_msg, perm)
    Gc = _blockdiag(rel_msg_cau, perm)
    G_p = G[jnp.asarray(EPERM)]
    Gc_p = Gc[jnp.asarray(EPERM)]
    te_hm = jnp.transpose(time_emb, (1, 0, 2)).reshape(128)
    te_dm = te_hm[perm][None, :]
    comb_hm = comb_pri.reshape(3, 128)
    comb_p = comb_hm[jnp.asarray(EPERM)][:, None, :]
    scales = jnp.stack([rel_pri[:, hol] / 4.0, rel_pri_cau[:, hol] / 4.0])

    ct = jnp.broadcast_to(cau_type[:, None], (N, 128)).astype(jnp.int32)

    ktab, vtab, qtab = _stage1(x[0], x[1], x[2], ct, A_k, b_k, A_v, b_v,
                               A_m, b_m, A_q, b_q)
    accn, accd = _sc_edge(ktab.reshape(2 * N, 128), vtab.reshape(2 * N, 128),
                          qtab.reshape(3 * N, 128),
                          src_idx.reshape(3 * E), dst_idx.reshape(3 * E),
                          scales.reshape(96))
    return _stage3(accn, accd, G_p, Gc_p, te_dm, comb_p)


# speedup vs baseline: 1.7793x; 1.0926x over previous
import functools

import jax
import jax.numpy as jnp
import numpy as np
from jax import lax
from jax.experimental import pallas as pl
from jax.experimental.pallas import tpu as pltpu
from jax.experimental.pallas import tpu_sc as plsc

N = 10000
E = 160000
D = 128
H = 8
DK = 16
DNT = (1, 0, 2)
EPERM = (1, 0, 2)

BLK = 400
NBLK = N // BLK

NTILE = 16
E_PER_TILE = E // NTILE
CHUNK = 48
NFULL = E_PER_TILE // CHUNK
TAIL = E_PER_TILE - NFULL * CHUNK
GROUP = 4
NGROUP = NFULL // GROUP
RCHUNK = 16
NRCH = N // RCHUNK
NRCH_PER_TILE = (NRCH + NTILE - 1) // NTILE


def _build_perm():
    perm = np.zeros(128, dtype=np.int32)
    for c in range(128):
        j, r = divmod(c, 16)
        p, hh = divmod(r, 8)
        h = hh if p == 0 else 7 - hh
        d = 2 * j + p
        perm[c] = h * 16 + d
    return perm


_PERM = _build_perm()
_HOL = np.array([r % 8 if r < 8 else 7 - (r % 8) for r in range(16)],
                dtype=np.int32)


def _stage1_body(x0, x1, x2, ct, A_k, b_k, A_v, b_v, A_m, b_m, A_q, b_q,
                 ktab, vtab, qtab):
    x0b = x0[...]
    kd = jnp.dot(x0b, A_k[...], preferred_element_type=jnp.float32) + b_k[...]
    vd = jnp.dot(x0b, A_v[...], preferred_element_type=jnp.float32) + b_v[...]
    ctb = ct[...]
    md = jnp.zeros_like(kd)
    for t in range(3):
        mm = jnp.dot(x0b, A_m[t], preferred_element_type=jnp.float32) + b_m[t, 0:1, :]
        md = md + jnp.where(ctb == t, mm, 0.0)
    ktab[0:1] = kd[None]
    ktab[1:2] = md[None]
    vtab[0:1] = vd[None]
    vtab[1:2] = vd[None]
    xs = (x1[...], x0b, x2[...])
    for e in range(3):
        q = jnp.dot(xs[e], A_q[e], preferred_element_type=jnp.float32) + b_q[e, 0:1, :]
        qtab[e:e + 1] = q[None]


def _stage1(x0, x1, x2, ct, A_k, b_k, A_v, b_v, A_m, b_m, A_q, b_q):
    full = lambda s: pl.BlockSpec(s, lambda i: (0,) * len(s))
    row = lambda s: pl.BlockSpec(s, lambda i: (i,) + (0,) * (len(s) - 1))
    return pl.pallas_call(
        _stage1_body,
        grid=(NBLK,),
        in_specs=[
            row((BLK, 128)), row((BLK, 128)), row((BLK, 128)), row((BLK, 128)),
            full((128, 128)), full((1, 128)),
            full((128, 128)), full((1, 128)),
            full((3, 128, 128)), full((3, 1, 128)),
            full((3, 128, 128)), full((3, 1, 128)),
        ],
        out_specs=[
            pl.BlockSpec((2, BLK, 128), lambda i: (0, i, 0)),
            pl.BlockSpec((2, BLK, 128), lambda i: (0, i, 0)),
            pl.BlockSpec((3, BLK, 128), lambda i: (0, i, 0)),
        ],
        out_shape=[
            jax.ShapeDtypeStruct((2, N, 128), jnp.float32),
            jax.ShapeDtypeStruct((2, N, 128), jnp.float32),
            jax.ShapeDtypeStruct((3, N, 128), jnp.float32),
        ],
    )(x0, x1, x2, ct, A_k, b_k, A_v, b_v, A_m, b_m, A_q, b_q)


def _sc_body(ktab_hbm, vtab_hbm, qtab_hbm, src_hbm, dst_hbm, scales_hbm,
             outn_hbm, outd_hbm,
             sidx0, dstv0, qidx0, kbuf0, qrows0, outbv0, exb0,
             sidx1, dstv1, qidx1, kbuf1, qrows1, outbv1, exb1,
             sblkA, dblkA, sblkB, dblkB,
             sidxt, dstvt, qidxt, scalev, sem0, sem1, ssem0, ssem1,
             numacc, denacc):
    br = lax.axis_index("c")
    tl = lax.axis_index("s")
    sets = ((sidx0, dstv0, qidx0, kbuf0, qrows0, outbv0, exb0, sem0, ssem0),
            (sidx1, dstv1, qidx1, kbuf1, qrows1, outbv1, exb1, sem1, ssem1))
    blks = ((sblkA, dblkA), (sblkB, dblkB))

    for e in range(3):
        def zrow(i, carry):
            r = i // 8
            col = (i % 8) * 16
            outbv0[r, pl.ds(col, 16)] = jnp.zeros((16,), jnp.float32)
            return carry
        lax.fori_loop(0, RCHUNK * 8, zrow, 0)

        def zrow2(i, carry):
            exb0[i, pl.ds(0, 16)] = jnp.zeros((16,), jnp.float32)
            return carry
        lax.fori_loop(0, RCHUNK, zrow2, 0)

        def zinit(i, carry):
            c = jnp.minimum(i * NTILE + tl, NRCH - 1)
            off = pl.multiple_of(c * RCHUNK, RCHUNK)
            pltpu.sync_copy(outbv0.at[pl.ds(0, RCHUNK)],
                            numacc.at[pl.ds(off, RCHUNK)])
            pltpu.sync_copy(exb0.at[pl.ds(0, RCHUNK)],
                            denacc.at[pl.ds(off, RCHUNK)])
            return carry
        lax.fori_loop(0, NRCH_PER_TILE, zinit, 0)
        soff = pl.multiple_of((br * 3 + e) * 16, 16)
        pltpu.sync_copy(scales_hbm.at[pl.ds(soff, 16)], scalev)
        plsc.subcore_barrier()

        def blkload(g, pair):
            gg = jnp.minimum(g, NGROUP - 1)
            off = pl.multiple_of(
                e * E + tl * E_PER_TILE + gg * (GROUP * CHUNK), 16)
            pltpu.sync_copy(src_hbm.at[pl.ds(off, GROUP * CHUNK)], pair[0])
            pltpu.sync_copy(dst_hbm.at[pl.ds(off, GROUP * CHUNK)], pair[1])

        def prep(j, s, pair, slot, first=False):
            sidx, dstv, qidx, kbuf, qrows, outbv, exb, sem, ssem = s
            if not first:
                @pl.when(j >= 2)
                def _():
                    pltpu.make_async_copy(outbv, numacc.at[dstv], ssem).wait()
                    pltpu.make_async_copy(exb, denacc.at[dstv], ssem).wait()
            sblk, dblk = pair
            for t in range(CHUNK // 16):
                col = slot * CHUNK + t * 16
                dv = dblk[pl.ds(col, 16)]
                sidx[pl.ds(t * 16, 16)] = sblk[pl.ds(col, 16)] + br * N
                qidx[pl.ds(t * 16, 16)] = dv + e * N
                dstv[pl.ds(t * 16, 16)] = dv
            pltpu.make_async_copy(vtab_hbm.at[sidx], outbv, sem).start()
            pltpu.make_async_copy(ktab_hbm.at[sidx], kbuf, sem).start()
            pltpu.make_async_copy(qtab_hbm.at[qidx], qrows, sem).start()

        def consume(s):
            sidx, dstv, qidx, kbuf, qrows, outbv, exb, sem, ssem = s
            pltpu.make_async_copy(vtab_hbm.at[sidx], outbv, sem).wait()
            pltpu.make_async_copy(ktab_hbm.at[sidx], kbuf, sem).wait()
            pltpu.make_async_copy(qtab_hbm.at[qidx], qrows, sem).wait()
            sc = scalev[...]

            def eb(c, c2):
                a = kbuf[c, pl.ds(0, 16)] * qrows[c, pl.ds(0, 16)]
                for j in range(1, 8):
                    a = a + kbuf[c, pl.ds(j * 16, 16)] * qrows[c, pl.ds(j * 16, 16)]
                a = a + lax.rev(a, (0,))
                ex = jnp.exp(jnp.minimum(a * sc, 60.0))
                exb[c, pl.ds(0, 16)] = ex
                for j in range(8):
                    outbv[c, pl.ds(j * 16, 16)] = ex * outbv[c, pl.ds(j * 16, 16)]
                return c2
            lax.fori_loop(0, CHUNK, eb, 0)
            pltpu.make_async_copy(outbv, numacc.at[dstv], ssem).start(add=True)
            pltpu.make_async_copy(exb, denacc.at[dstv], ssem).start(add=True)

        blkload(0, blks[0])
        prep(0, sets[0], blks[0], 0, first=True)

        @pl.loop(0, NGROUP, step=2)
        def _ring(go):
            for gp in range(2):
                gi = go + gp
                cur = blks[gp]
                nxt = blks[1 - gp]
                blkload(gi + 1, nxt)
                for b in range(GROUP):
                    j = gi * GROUP + b
                    s = sets[b % 2]
                    npair = cur if b < GROUP - 1 else nxt

                    @pl.when(j + 1 < NFULL)
                    def _(npair=npair, b=b, j=j):
                        prep(j + 1, sets[(b + 1) % 2], npair, (b + 1) % GROUP)
                    consume(s)

        for s in sets:
            _, dstv_s, _, _, _, outbv_s, exb_s, _, ssem_s = s
            pltpu.make_async_copy(outbv_s, numacc.at[dstv_s], ssem_s).wait()
            pltpu.make_async_copy(exb_s, denacc.at[dstv_s], ssem_s).wait()

        toff = pl.multiple_of(e * E + tl * E_PER_TILE + NFULL * CHUNK, 16)
        pltpu.sync_copy(src_hbm.at[pl.ds(toff, TAIL)], sidxt)
        pltpu.sync_copy(dst_hbm.at[pl.ds(toff, TAIL)], dstvt)
        sidxt[pl.ds(0, 16)] = sidxt[pl.ds(0, 16)] + br * N
        qidxt[pl.ds(0, 16)] = dstvt[pl.ds(0, 16)] + e * N
        pltpu.sync_copy(vtab_hbm.at[sidxt], outbv0.at[pl.ds(0, TAIL)])
        pltpu.sync_copy(ktab_hbm.at[sidxt], kbuf0.at[pl.ds(0, TAIL)])
        pltpu.sync_copy(qtab_hbm.at[qidxt], qrows0.at[pl.ds(0, TAIL)])
        sct = scalev[...]

        def ebt(c, c2):
            a = kbuf0[c, pl.ds(0, 16)] * qrows0[c, pl.ds(0, 16)]
            for j in range(1, 8):
                a = a + kbuf0[c, pl.ds(j * 16, 16)] * qrows0[c, pl.ds(j * 16, 16)]
            a = a + lax.rev(a, (0,))
            ex = jnp.exp(jnp.minimum(a * sct, 60.0))
            exb0[c, pl.ds(0, 16)] = ex
            for j in range(8):
                outbv0[c, pl.ds(j * 16, 16)] = ex * outbv0[c, pl.ds(j * 16, 16)]
            return c2
        lax.fori_loop(0, TAIL, ebt, 0)
        pltpu.sync_copy(outbv0.at[pl.ds(0, TAIL)], numacc.at[dstvt], add=True)
        pltpu.sync_copy(exb0.at[pl.ds(0, TAIL)], denacc.at[dstvt], add=True)
        plsc.subcore_barrier()

        def cout(i, carry):
            c = jnp.minimum(i * NTILE + tl, NRCH - 1)
            off = pl.multiple_of(c * RCHUNK, RCHUNK)
            pltpu.sync_copy(numacc.at[pl.ds(off, RCHUNK)],
                            outn_hbm.at[br, EPERM[e], pl.ds(off, RCHUNK)])
            pltpu.sync_copy(denacc.at[pl.ds(off, RCHUNK)],
                            outd_hbm.at[br, EPERM[e], pl.ds(off, RCHUNK)])
            return carry
        lax.fori_loop(0, NRCH_PER_TILE, cout, 0)
        plsc.subcore_barrier()


@functools.partial(jax.jit, static_argnums=())
def _sc_edge(ktab, vtab, qtab, src, dst, scales):
    mesh = plsc.VectorSubcoreMesh(core_axis_name="c", subcore_axis_name="s")
    bufset = [
        pltpu.VMEM((CHUNK,), jnp.int32),
        pltpu.VMEM((CHUNK,), jnp.int32),
        pltpu.VMEM((CHUNK,), jnp.int32),
        pltpu.VMEM((CHUNK, 128), jnp.float32),
        pltpu.VMEM((CHUNK, 128), jnp.float32),
        pltpu.VMEM((CHUNK, 128), jnp.float32),
        pltpu.VMEM((CHUNK, 16), jnp.float32),
    ]
    f = pl.kernel(
        _sc_body,
        out_type=[
            jax.ShapeDtypeStruct((2, 3, N, 128), jnp.float32),
            jax.ShapeDtypeStruct((2, 3, N, 16), jnp.float32),
        ],
        mesh=mesh,
        compiler_params=pltpu.CompilerParams(use_tc_tiling_on_sc=False),
        scratch_types=bufset + bufset + [
            pltpu.VMEM((GROUP * CHUNK,), jnp.int32),
            pltpu.VMEM((GROUP * CHUNK,), jnp.int32),
            pltpu.VMEM((GROUP * CHUNK,), jnp.int32),
            pltpu.VMEM((GROUP * CHUNK,), jnp.int32),
            pltpu.VMEM((TAIL,), jnp.int32),
            pltpu.VMEM((TAIL,), jnp.int32),
            pltpu.VMEM((TAIL,), jnp.int32),
            pltpu.VMEM((16,), jnp.float32),
            pltpu.SemaphoreType.DMA,
            pltpu.SemaphoreType.DMA,
            pltpu.SemaphoreType.DMA,
            pltpu.SemaphoreType.DMA,
            pltpu.VMEM_SHARED((N, 128), jnp.float32),
            pltpu.VMEM_SHARED((N, 16), jnp.float32),
        ],
    )
    return f(ktab, vtab, qtab, src, dst, scales)


def _stage3_body(accn_r, accd_r, G, Gc, te, comb, out):
    num = accn_r[0, 0]
    den16 = accd_r[0, 0]
    den = jnp.tile(den16, (1, 8))
    den = jnp.where(den == 0.0, 1.0, den)
    h = jnp.dot(num / den, G[0], preferred_element_type=jnp.float32)
    cnum = accn_r[1, 0]
    cden16 = accd_r[1, 0]
    cden = jnp.tile(cden16, (1, 8))
    cdeng = jnp.where(cden == 0.0, 1.0, cden)
    teb = te[...]
    ch = jnp.dot((cnum + cden * teb) / cdeng, Gc[0],
                 preferred_element_type=jnp.float32)
    out[0] = jnp.maximum(h + ch * comb[0], 0.0)


def _stage3(accn, accd, G_p, Gc_p, te_dm, comb_p):
    return pl.pallas_call(
        _stage3_body,
        grid=(3, NBLK),
        in_specs=[
            pl.BlockSpec((2, 1, BLK, 128), lambda e, i: (0, e, i, 0)),
            pl.BlockSpec((2, 1, BLK, 16), lambda e, i: (0, e, i, 0)),
            pl.BlockSpec((1, 128, 128), lambda e, i: (e, 0, 0)),
            pl.BlockSpec((1, 128, 128), lambda e, i: (e, 0, 0)),
            pl.BlockSpec((1, 128), lambda e, i: (0, 0)),
            pl.BlockSpec((1, 1, 128), lambda e, i: (e, 0, 0)),
        ],
        out_specs=pl.BlockSpec((1, BLK, 128), lambda e, i: (e, i, 0)),
        out_shape=jax.ShapeDtypeStruct((3, N, 128), jnp.float32),
    )(accn, accd, G_p, Gc_p, te_dm, comb_p)


def _blockdiag(rel, perm):
    out = jnp.zeros((3, 128, 128), dtype=jnp.float32)
    for e in range(3):
        for h in range(H):
            out = out.at[e, h * 16:(h + 1) * 16, h * 16:(h + 1) * 16].set(rel[e, h])
    return out[:, perm, :]


def kernel(x, Wk, bk, Wq, bq, Wv, bv, rel_pri, rel_msg, rel_pri_cau,
           rel_msg_cau, comb_pri, cau_filter, time_emb, src_idx, dst_idx,
           cau_type):
    perm = jnp.asarray(_PERM)
    hol = jnp.asarray(_HOL)

    A_k = Wk[0].T[:, perm]
    b_k = bk[0][perm][None, :]
    A_v = Wv[0].T[:, perm]
    b_v = bv[0][perm][None, :]
    Mbd = _blockdiag(cau_filter, jnp.arange(128))
    A_m = jnp.einsum('ab,tbc->tac', Wk[0].T, Mbd[:, :, perm])
    b_m = jnp.einsum('b,tbc->tc', bk[0], Mbd[:, :, perm])[:, None, :]
    A_q = jnp.stack([Wq[DNT[e]].T[:, perm] for e in range(3)])
    b_q = jnp.stack([bq[DNT[e]][perm] for e in range(3)])[:, None, :]
    G = _blockdiag(rel_msg, perm)
    Gc = _blockdiag(rel_msg_cau, perm)
    G_p = G[jnp.asarray(EPERM)]
    Gc_p = Gc[jnp.asarray(EPERM)]
    te_hm = jnp.transpose(time_emb, (1, 0, 2)).reshape(128)
    te_dm = te_hm[perm][None, :]
    comb_hm = comb_pri.reshape(3, 128)
    comb_p = comb_hm[jnp.asarray(EPERM)][:, None, :]
    scales = jnp.stack([rel_pri[:, hol] / 4.0, rel_pri_cau[:, hol] / 4.0])

    ct = jnp.broadcast_to(cau_type[:, None], (N, 128)).astype(jnp.int32)

    ktab, vtab, qtab = _stage1(x[0], x[1], x[2], ct, A_k, b_k, A_v, b_v,
                               A_m, b_m, A_q, b_q)
    accn, accd = _sc_edge(ktab.reshape(2 * N, 128), vtab.reshape(2 * N, 128),
                          qtab.reshape(3 * N, 128),
                          src_idx.reshape(3 * E), dst_idx.reshape(3 * E),
                          scales.reshape(96))
    return _stage3(accn, accd, G_p, Gc_p, te_dm, comb_p)
